# baseline (speedup 1.0000x reference)
import functools

import jax
import jax.numpy as jnp
from jax import lax
from jax.experimental import pallas as pl
from jax.experimental.pallas import tpu as pltpu
from jax.experimental.pallas import tpu_sc as plsc

N = 10000
E = 320000
D = 128
B = 128
C = 10

NC = 2
NS = 16
NW = NC * NS
EPT = 10240
E_PAD = NW * EPT
N_PAD = 10240
RPT = N_PAD // NS


def _mesh():
    return plsc.VectorSubcoreMesh(core_axis_name="c", subcore_axis_name="s")


_DEG_K, _DEG_NCH = 2560, 4


def _deg_body(dst_hbm, zeros_hbm, ones_hbm, out_hbm, *refs):
    idst = refs[:_DEG_NCH]
    ones_v = refs[_DEG_NCH]
    deg_sh = refs[_DEG_NCH + 1]
    ssem = refs[_DEG_NCH + 2]
    c = lax.axis_index("c")
    s = lax.axis_index("s")
    w = c * NS + s
    pltpu.sync_copy(zeros_hbm.at[pl.ds(s * RPT, RPT)],
                    deg_sh.at[pl.ds(s * RPT, RPT)])
    pltpu.sync_copy(ones_hbm, ones_v)
    for ch in range(_DEG_NCH):
        pltpu.sync_copy(dst_hbm.at[w, ch], idst[ch])
    plsc.subcore_barrier()
    for ch in range(_DEG_NCH):
        pltpu.async_copy(ones_v, deg_sh.at[idst[ch]], ssem, add=True)
    for ch in range(_DEG_NCH):
        pltpu.make_async_copy(ones_v, deg_sh.at[idst[ch]], ssem).wait()
    plsc.subcore_barrier()
    pltpu.sync_copy(deg_sh.at[pl.ds(s * RPT, RPT)],
                    out_hbm.at[c, pl.ds(s * RPT, RPT)])


_deg_sc = pl.kernel(
    _deg_body,
    out_type=jax.ShapeDtypeStruct((NC, N_PAD, 16), jnp.float32),
    mesh=_mesh(),
    scratch_types=(
        [pltpu.VMEM((_DEG_K,), jnp.int32) for _ in range(_DEG_NCH)]
        + [pltpu.VMEM((_DEG_K, 16), jnp.float32),
           pltpu.VMEM_SHARED((N_PAD, 16), jnp.float32),
           pltpu.SemaphoreType.DMA]
    ),
    compiler_params=pltpu.CompilerParams(use_tc_tiling_on_sc=False),
)


def _agg_body(F, K, NCH, h_hbm, src_hbm, dst_hbm, zeros_hbm, out_hbm, *refs):
    isrc = refs[:NCH]
    idst = refs[NCH:2 * NCH]
    rows = refs[2 * NCH:2 * NCH + 2]
    agg_sh = refs[2 * NCH + 2]
    gsem = refs[2 * NCH + 3:2 * NCH + 5]
    ssem = refs[2 * NCH + 5:2 * NCH + 7]
    c = lax.axis_index("c")
    s = lax.axis_index("s")
    w = c * NS + s
    pltpu.sync_copy(zeros_hbm.at[pl.ds(s * RPT, RPT)],
                    agg_sh.at[pl.ds(s * RPT, RPT)])
    for ch in range(NCH):
        pltpu.sync_copy(src_hbm.at[w, ch], isrc[ch])
        pltpu.sync_copy(dst_hbm.at[w, ch], idst[ch])
    plsc.subcore_barrier()

    def sg(ch, b):
        pltpu.async_copy(h_hbm.at[isrc[ch]], rows[b], gsem[b])

    def wg(ch, b):
        pltpu.make_async_copy(h_hbm.at[isrc[ch]], rows[b], gsem[b]).wait()

    def ss(ch, b):
        pltpu.async_copy(rows[b], agg_sh.at[idst[ch]], ssem[b], add=True)

    def ws(ch, b):
        pltpu.make_async_copy(rows[b], agg_sh.at[idst[ch]], ssem[b]).wait()

    sg(0, 0)
    for ch in range(NCH):
        b = ch % 2
        nb = (ch + 1) % 2
        wg(ch, b)
        ss(ch, b)
        if ch + 1 < NCH:
            if ch >= 1:
                ws(ch - 1, nb)
            sg(ch + 1, nb)
    if NCH >= 2:
        ws(NCH - 2, (NCH - 2) % 2)
    ws(NCH - 1, (NCH - 1) % 2)
    plsc.subcore_barrier()
    pltpu.sync_copy(agg_sh.at[pl.ds(s * RPT, RPT)],
                    out_hbm.at[c, pl.ds(s * RPT, RPT)])


def _make_agg(F, K, NCH):
    assert K * NCH == EPT
    return pl.kernel(
        functools.partial(_agg_body, F, K, NCH),
        out_type=jax.ShapeDtypeStruct((NC, N_PAD, F), jnp.float32),
        mesh=_mesh(),
        scratch_types=(
            [pltpu.VMEM((K,), jnp.int32) for _ in range(2 * NCH)]
            + [pltpu.VMEM((K, F), jnp.float32) for _ in range(2)]
            + [pltpu.VMEM_SHARED((N_PAD, F), jnp.float32)]
            + [pltpu.SemaphoreType.DMA for _ in range(4)]
        ),
        compiler_params=pltpu.CompilerParams(use_tc_tiling_on_sc=False),
    )


_agg16 = _make_agg(16, 2560, 4)
_agg32 = _make_agg(32, 1280, 8)


def _tc1_body(x_ref, w1_ref, degp_ref, h1s_ref, dinv_ref):
    deg = degp_ref[0][:, :1] + degp_ref[1][:, :1] + 1.0
    dinv = lax.rsqrt(deg)
    dinv_ref[...] = dinv
    h = jnp.dot(x_ref[...], w1_ref[...], preferred_element_type=jnp.float32)
    h1s_ref[...] = h * dinv[:N]


def _tc2_body(agg_ref, h1s_ref, dinv_ref, b1_ref, w2_ref, h2s_ref):
    dinv = dinv_ref[...][:N]
    z = (agg_ref[0][:N] + agg_ref[1][:N] + h1s_ref[...]) * dinv + b1_ref[...]
    z = jnp.maximum(z, 0.0)
    h2s_ref[...] = jnp.dot(z, w2_ref[...],
                           preferred_element_type=jnp.float32) * dinv


def _tc3_body(agg_ref, h2s_ref, dinv_ref, b2_ref, batch_ref,
              wfc1_ref, bfc1_ref, wfc2_ref, bfc2_ref, out_ref):
    dinv = dinv_ref[...][:N]
    z = (agg_ref[0][:N] + agg_ref[1][:N] + h2s_ref[...]) * dinv + b2_ref[...]
    z = jnp.maximum(z, 0.0)
    oh = (batch_ref[...] == lax.broadcasted_iota(jnp.int32, (B, N), 0))
    oh = oh.astype(jnp.float32)
    sums = jnp.dot(oh, z, preferred_element_type=jnp.float32)
    counts = jnp.sum(oh, axis=1, keepdims=True)
    pooled = sums / jnp.maximum(counts, 1.0)
    hfc = jnp.maximum(
        jnp.dot(pooled, wfc1_ref[...], preferred_element_type=jnp.float32)
        + bfc1_ref[...], 0.0)
    out_ref[...] = jnp.dot(hfc, wfc2_ref[...],
                           preferred_element_type=jnp.float32) + bfc2_ref[...]


_tc1 = pl.pallas_call(
    _tc1_body,
    out_shape=(jax.ShapeDtypeStruct((N, 16), jnp.float32),
               jax.ShapeDtypeStruct((N_PAD, 1), jnp.float32)))

_tc2 = pl.pallas_call(
    _tc2_body, out_shape=jax.ShapeDtypeStruct((N, 32), jnp.float32))

_tc3 = pl.pallas_call(
    _tc3_body, out_shape=jax.ShapeDtypeStruct((B, C), jnp.float32))


def kernel(x, edge_index, batch, W1, b1, W2, b2, Wfc1, bfc1, Wfc2, bfc2):
    src = edge_index[0]
    dst = edge_index[1]
    pad = E_PAD - E
    src_p = jnp.concatenate([src, jnp.zeros((pad,), jnp.int32)])
    dst_p = jnp.concatenate([dst, jnp.full((pad,), N, jnp.int32)])

    zeros16 = jnp.zeros((N_PAD, 16), jnp.float32)
    ones16 = jnp.ones((_DEG_K, 16), jnp.float32)
    degp = _deg_sc(dst_p.reshape(NW, _DEG_NCH, _DEG_K), zeros16, ones16)

    h1s, dinv = _tc1(x, W1, degp)

    agg1 = _agg16(h1s, src_p.reshape(NW, 4, 2560),
                  dst_p.reshape(NW, 4, 2560), zeros16)

    h2s = _tc2(agg1, h1s, dinv, b1.reshape(1, 16), W2)

    zeros32 = jnp.zeros((N_PAD, 32), jnp.float32)
    agg2 = _agg32(h2s, src_p.reshape(NW, 8, 1280),
                  dst_p.reshape(NW, 8, 1280), zeros32)

    return _tc3(agg2, h2s, dinv, b2.reshape(1, 32), batch.reshape(1, N),
                Wfc1, bfc1.reshape(1, 64), Wfc2, bfc2.reshape(1, C))


# speedup vs baseline: 1.2834x; 1.2834x over previous
import functools

import jax
import jax.numpy as jnp
from jax import lax
from jax.experimental import pallas as pl
from jax.experimental.pallas import tpu as pltpu
from jax.experimental.pallas import tpu_sc as plsc

N = 10000
E = 320000
D = 128
B = 128
C = 10

NC = 2
NS = 16
NW = NC * NS
EPT = 10240
E_PAD = NW * EPT
N_PAD = 10240
RPT = N_PAD // NS


def _mesh():
    return plsc.VectorSubcoreMesh(core_axis_name="c", subcore_axis_name="s")


_DEG_K, _DEG_NCH = 2560, 4


def _deg_body(dst_hbm, zeros_hbm, ones_hbm, out_hbm, *refs):
    idst = refs[:_DEG_NCH]
    ones_v = refs[_DEG_NCH]
    deg_sh = refs[_DEG_NCH + 1]
    ssem = refs[_DEG_NCH + 2]
    c = lax.axis_index("c")
    s = lax.axis_index("s")
    w = c * NS + s
    pltpu.sync_copy(zeros_hbm.at[pl.ds(s * RPT, RPT)],
                    deg_sh.at[pl.ds(s * RPT, RPT)])
    pltpu.sync_copy(ones_hbm, ones_v)
    for ch in range(_DEG_NCH):
        pltpu.sync_copy(dst_hbm.at[w, ch], idst[ch])
    plsc.subcore_barrier()
    for ch in range(_DEG_NCH):
        pltpu.async_copy(ones_v, deg_sh.at[idst[ch]], ssem, add=True)
    for ch in range(_DEG_NCH):
        pltpu.make_async_copy(ones_v, deg_sh.at[idst[ch]], ssem).wait()
    plsc.subcore_barrier()
    pltpu.sync_copy(deg_sh.at[pl.ds(s * RPT, RPT)],
                    out_hbm.at[c, pl.ds(s * RPT, RPT)])


_deg_sc = pl.kernel(
    _deg_body,
    out_type=jax.ShapeDtypeStruct((NC, N_PAD, 16), jnp.float32),
    mesh=_mesh(),
    scratch_types=(
        [pltpu.VMEM((_DEG_K,), jnp.int32) for _ in range(_DEG_NCH)]
        + [pltpu.VMEM((_DEG_K, 16), jnp.float32),
           pltpu.VMEM_SHARED((N_PAD, 16), jnp.float32),
           pltpu.SemaphoreType.DMA]
    ),
    compiler_params=pltpu.CompilerParams(use_tc_tiling_on_sc=False),
)


def _agg_body(F, K, NCH, h_hbm, src_hbm, dst_hbm, zeros_hbm, out_hbm, *refs):
    isrc = refs[:NCH]
    idst = refs[NCH:2 * NCH]
    rows = refs[2 * NCH:2 * NCH + 2]
    agg_sh = refs[2 * NCH + 2]
    gsem = refs[2 * NCH + 3:2 * NCH + 5]
    ssem = refs[2 * NCH + 5:2 * NCH + 7]
    c = lax.axis_index("c")
    s = lax.axis_index("s")
    w = c * NS + s
    pltpu.sync_copy(zeros_hbm.at[pl.ds(s * RPT, RPT)],
                    agg_sh.at[pl.ds(s * RPT, RPT)])
    for ch in range(NCH):
        pltpu.sync_copy(src_hbm.at[w, ch], isrc[ch])
        pltpu.sync_copy(dst_hbm.at[w, ch], idst[ch])
    plsc.subcore_barrier()

    def sg(ch, b):
        pltpu.async_copy(h_hbm.at[isrc[ch]], rows[b], gsem[b])

    def wg(ch, b):
        pltpu.make_async_copy(h_hbm.at[isrc[ch]], rows[b], gsem[b]).wait()

    def ss(ch, b):
        pltpu.async_copy(rows[b], agg_sh.at[idst[ch]], ssem[b], add=True)

    def ws(ch, b):
        pltpu.make_async_copy(rows[b], agg_sh.at[idst[ch]], ssem[b]).wait()

    sg(0, 0)
    for ch in range(NCH):
        b = ch % 2
        nb = (ch + 1) % 2
        wg(ch, b)
        ss(ch, b)
        if ch + 1 < NCH:
            if ch >= 1:
                ws(ch - 1, nb)
            sg(ch + 1, nb)
    if NCH >= 2:
        ws(NCH - 2, (NCH - 2) % 2)
    ws(NCH - 1, (NCH - 1) % 2)
    plsc.subcore_barrier()
    pltpu.sync_copy(agg_sh.at[pl.ds(s * RPT, RPT)],
                    out_hbm.at[c, pl.ds(s * RPT, RPT)])


def _make_agg(F, K, NCH):
    assert K * NCH == EPT
    return pl.kernel(
        functools.partial(_agg_body, F, K, NCH),
        out_type=jax.ShapeDtypeStruct((NC, N_PAD, F), jnp.float32),
        mesh=_mesh(),
        scratch_types=(
            [pltpu.VMEM((K,), jnp.int32) for _ in range(2 * NCH)]
            + [pltpu.VMEM((K, F), jnp.float32) for _ in range(2)]
            + [pltpu.VMEM_SHARED((N_PAD, F), jnp.float32)]
            + [pltpu.SemaphoreType.DMA for _ in range(4)]
        ),
        compiler_params=pltpu.CompilerParams(use_tc_tiling_on_sc=False),
    )


_agg16 = _make_agg(16, 2560, 4)


def _tc1_body(x_ref, w1_ref, degp_ref, h1s_ref, dinv_ref):
    deg = degp_ref[0][:, :1] + degp_ref[1][:, :1] + 1.0
    dinv = lax.rsqrt(deg)
    dinv_ref[...] = dinv
    h = jnp.dot(x_ref[...], w1_ref[...], preferred_element_type=jnp.float32)
    h1s_ref[...] = h * dinv[:N]


def _tc2_body(agg_ref, h1s_ref, dinv_ref, b1_ref, q_ref):
    dinv = dinv_ref[...][:N]
    z = (agg_ref[0][:N] + agg_ref[1][:N] + h1s_ref[...]) * dinv + b1_ref[...]
    z = jnp.maximum(z, 0.0)
    q_ref[...] = z * dinv


def _tc3_body(agg_ref, q_ref, dinv_ref, b2_ref, w2_ref, batch_ref,
              wfc1_ref, bfc1_ref, wfc2_ref, bfc2_ref, out_ref):
    dinv = dinv_ref[...][:N]
    t = (agg_ref[0][:N] + agg_ref[1][:N] + q_ref[...]) * dinv
    z = jnp.dot(t, w2_ref[...],
                preferred_element_type=jnp.float32) + b2_ref[...]
    z = jnp.maximum(z, 0.0)
    oh = (batch_ref[...] == lax.broadcasted_iota(jnp.int32, (B, N), 0))
    oh = oh.astype(jnp.float32)
    sums = jnp.dot(oh, z, preferred_element_type=jnp.float32)
    counts = jnp.sum(oh, axis=1, keepdims=True)
    pooled = sums / jnp.maximum(counts, 1.0)
    hfc = jnp.maximum(
        jnp.dot(pooled, wfc1_ref[...], preferred_element_type=jnp.float32)
        + bfc1_ref[...], 0.0)
    out_ref[...] = jnp.dot(hfc, wfc2_ref[...],
                           preferred_element_type=jnp.float32) + bfc2_ref[...]


_tc1 = pl.pallas_call(
    _tc1_body,
    out_shape=(jax.ShapeDtypeStruct((N, 16), jnp.float32),
               jax.ShapeDtypeStruct((N_PAD, 1), jnp.float32)))

_tc2 = pl.pallas_call(
    _tc2_body, out_shape=jax.ShapeDtypeStruct((N, 16), jnp.float32))

_tc3 = pl.pallas_call(
    _tc3_body, out_shape=jax.ShapeDtypeStruct((B, C), jnp.float32))


def kernel(x, edge_index, batch, W1, b1, W2, b2, Wfc1, bfc1, Wfc2, bfc2):
    src = edge_index[0]
    dst = edge_index[1]
    pad = E_PAD - E
    src_p = jnp.concatenate([src, jnp.zeros((pad,), jnp.int32)])
    dst_p = jnp.concatenate([dst, jnp.full((pad,), N, jnp.int32)])

    zeros16 = jnp.zeros((N_PAD, 16), jnp.float32)
    ones16 = jnp.ones((_DEG_K, 16), jnp.float32)
    degp = _deg_sc(dst_p.reshape(NW, _DEG_NCH, _DEG_K), zeros16, ones16)

    h1s, dinv = _tc1(x, W1, degp)

    agg1 = _agg16(h1s, src_p.reshape(NW, 4, 2560),
                  dst_p.reshape(NW, 4, 2560), zeros16)

    q = _tc2(agg1, h1s, dinv, b1.reshape(1, 16))

    agg2 = _agg16(q, src_p.reshape(NW, 4, 2560),
                  dst_p.reshape(NW, 4, 2560), zeros16)

    return _tc3(agg2, q, dinv, b2.reshape(1, 32), W2, batch.reshape(1, N),
                Wfc1, bfc1.reshape(1, 64), Wfc2, bfc2.reshape(1, C))


# speedup vs baseline: 1.6067x; 1.2520x over previous
import functools

import jax
import jax.numpy as jnp
from jax import lax
from jax.experimental import pallas as pl
from jax.experimental.pallas import tpu as pltpu
from jax.experimental.pallas import tpu_sc as plsc

N = 10000
E = 320000
D = 128
B = 128
C = 10

NC = 2
NS = 16
NW = NC * NS
K_EDGE = 1000
AGG_NCH0, AGG_NCH1 = 14, 6
DEG_NCH0, DEG_NCH1 = 11, 9
E_CORE0 = NS * AGG_NCH0 * K_EDGE
E_DEG0 = NS * DEG_NCH0 * K_EDGE
N_PAD = 10240
RPT = N_PAD // NS


def _mesh():
    return plsc.VectorSubcoreMesh(core_axis_name="c", subcore_axis_name="s")


def _deg_body(dst_hbm, zeros_hbm, ones_hbm, out_hbm, *refs):
    nmax = max(DEG_NCH0, DEG_NCH1)
    idst = refs[:nmax]
    ones_v = refs[nmax]
    deg_sh = refs[nmax + 1]
    ssem = refs[nmax + 2]
    c = lax.axis_index("c")
    s = lax.axis_index("s")
    pltpu.sync_copy(zeros_hbm.at[pl.ds(s * RPT, RPT)],
                    deg_sh.at[pl.ds(s * RPT, RPT)])
    pltpu.sync_copy(ones_hbm, ones_v)

    def load(nch, base):
        for ch in range(nch):
            pltpu.sync_copy(
                dst_hbm.at[pl.ds(base + ch * K_EDGE, K_EDGE)], idst[ch])

    def scat(nch):
        for ch in range(nch):
            pltpu.async_copy(ones_v, deg_sh.at[idst[ch]], ssem, add=True)
        for ch in range(nch):
            pltpu.make_async_copy(ones_v, deg_sh.at[idst[ch]], ssem).wait()

    @pl.when(c == 0)
    def _():
        load(DEG_NCH0, pl.multiple_of(s * (DEG_NCH0 * K_EDGE), 8))

    @pl.when(c == 1)
    def _():
        load(DEG_NCH1, pl.multiple_of(E_DEG0 + s * (DEG_NCH1 * K_EDGE), 8))

    plsc.subcore_barrier()

    @pl.when(c == 0)
    def _():
        scat(DEG_NCH0)

    @pl.when(c == 1)
    def _():
        scat(DEG_NCH1)

    plsc.subcore_barrier()
    pltpu.sync_copy(deg_sh.at[pl.ds(s * RPT, RPT)],
                    out_hbm.at[c, pl.ds(s * RPT, RPT)])


_deg_sc = pl.kernel(
    _deg_body,
    out_type=jax.ShapeDtypeStruct((NC, N_PAD, 16), jnp.float32),
    mesh=_mesh(),
    scratch_types=(
        [pltpu.VMEM((K_EDGE,), jnp.int32)
         for _ in range(max(DEG_NCH0, DEG_NCH1))]
        + [pltpu.VMEM((K_EDGE, 16), jnp.float32),
           pltpu.VMEM_SHARED((N_PAD, 16), jnp.float32),
           pltpu.SemaphoreType.DMA]
    ),
    compiler_params=pltpu.CompilerParams(use_tc_tiling_on_sc=False),
)


def _agg_body(F, h_hbm, src_hbm, dst_hbm, zeros_hbm, out_hbm, *refs):
    nmax = max(AGG_NCH0, AGG_NCH1)
    isrc = refs[:nmax]
    idst = refs[nmax:2 * nmax]
    rows = refs[2 * nmax:2 * nmax + 2]
    agg_sh = refs[2 * nmax + 2]
    gsem = refs[2 * nmax + 3:2 * nmax + 5]
    ssem = refs[2 * nmax + 5:2 * nmax + 7]
    c = lax.axis_index("c")
    s = lax.axis_index("s")
    pltpu.sync_copy(zeros_hbm.at[pl.ds(s * RPT, RPT)],
                    agg_sh.at[pl.ds(s * RPT, RPT)])

    def load(nch, base):
        for ch in range(nch):
            pltpu.sync_copy(
                src_hbm.at[pl.ds(base + ch * K_EDGE, K_EDGE)], isrc[ch])
            pltpu.sync_copy(
                dst_hbm.at[pl.ds(base + ch * K_EDGE, K_EDGE)], idst[ch])

    def sg(ch, b):
        pltpu.async_copy(h_hbm.at[isrc[ch]], rows[b], gsem[b])

    def wg(ch, b):
        pltpu.make_async_copy(h_hbm.at[isrc[ch]], rows[b], gsem[b]).wait()

    def ss(ch, b):
        pltpu.async_copy(rows[b], agg_sh.at[idst[ch]], ssem[b], add=True)

    def ws(ch, b):
        pltpu.make_async_copy(rows[b], agg_sh.at[idst[ch]], ssem[b]).wait()

    def pipeline(nch):
        sg(0, 0)
        for ch in range(nch):
            b = ch % 2
            nb = (ch + 1) % 2
            wg(ch, b)
            ss(ch, b)
            if ch + 1 < nch:
                if ch >= 1:
                    ws(ch - 1, nb)
                sg(ch + 1, nb)
        if nch >= 2:
            ws(nch - 2, (nch - 2) % 2)
        ws(nch - 1, (nch - 1) % 2)

    @pl.when(c == 0)
    def _():
        load(AGG_NCH0, pl.multiple_of(s * (AGG_NCH0 * K_EDGE), 8))

    @pl.when(c == 1)
    def _():
        load(AGG_NCH1, pl.multiple_of(E_CORE0 + s * (AGG_NCH1 * K_EDGE), 8))

    plsc.subcore_barrier()

    @pl.when(c == 0)
    def _():
        pipeline(AGG_NCH0)

    @pl.when(c == 1)
    def _():
        pipeline(AGG_NCH1)

    plsc.subcore_barrier()
    pltpu.sync_copy(agg_sh.at[pl.ds(s * RPT, RPT)],
                    out_hbm.at[c, pl.ds(s * RPT, RPT)])


def _make_agg(F):
    nmax = max(AGG_NCH0, AGG_NCH1)
    return pl.kernel(
        functools.partial(_agg_body, F),
        out_type=jax.ShapeDtypeStruct((NC, N_PAD, F), jnp.float32),
        mesh=_mesh(),
        scratch_types=(
            [pltpu.VMEM((K_EDGE,), jnp.int32) for _ in range(2 * nmax)]
            + [pltpu.VMEM((K_EDGE, F), jnp.float32) for _ in range(2)]
            + [pltpu.VMEM_SHARED((N_PAD, F), jnp.float32)]
            + [pltpu.SemaphoreType.DMA for _ in range(4)]
        ),
        compiler_params=pltpu.CompilerParams(use_tc_tiling_on_sc=False),
    )


_agg16 = _make_agg(16)


def _tc1_body(x_ref, w1_ref, degp_ref, h1s_ref, dinv_ref):
    deg = degp_ref[0][:, :1] + degp_ref[1][:, :1] + 1.0
    dinv = lax.rsqrt(deg)
    dinv_ref[...] = dinv
    h = jnp.dot(x_ref[...], w1_ref[...], preferred_element_type=jnp.float32)
    h1s_ref[...] = h * dinv[:N]


def _tc2_body(agg_ref, h1s_ref, dinv_ref, b1_ref, q_ref):
    dinv = dinv_ref[...][:N]
    z = (agg_ref[0][:N] + agg_ref[1][:N] + h1s_ref[...]) * dinv + b1_ref[...]
    z = jnp.maximum(z, 0.0)
    q_ref[...] = z * dinv


def _tc3_body(agg_ref, q_ref, dinv_ref, b2_ref, w2_ref, batch_ref,
              wfc1_ref, bfc1_ref, wfc2_ref, bfc2_ref, out_ref):
    dinv = dinv_ref[...][:N]
    t = (agg_ref[0][:N] + agg_ref[1][:N] + q_ref[...]) * dinv
    z = jnp.dot(t, w2_ref[...],
                preferred_element_type=jnp.float32) + b2_ref[...]
    z = jnp.maximum(z, 0.0)
    oh = (batch_ref[...] == lax.broadcasted_iota(jnp.int32, (B, N), 0))
    oh = oh.astype(jnp.float32)
    sums = jnp.dot(oh, z, preferred_element_type=jnp.float32)
    counts = jnp.sum(oh, axis=1, keepdims=True)
    pooled = sums / jnp.maximum(counts, 1.0)
    hfc = jnp.maximum(
        jnp.dot(pooled, wfc1_ref[...], preferred_element_type=jnp.float32)
        + bfc1_ref[...], 0.0)
    out_ref[...] = jnp.dot(hfc, wfc2_ref[...],
                           preferred_element_type=jnp.float32) + bfc2_ref[...]


_tc1 = pl.pallas_call(
    _tc1_body,
    out_shape=(jax.ShapeDtypeStruct((N, 16), jnp.float32),
               jax.ShapeDtypeStruct((N_PAD, 1), jnp.float32)))

_tc2 = pl.pallas_call(
    _tc2_body, out_shape=jax.ShapeDtypeStruct((N, 16), jnp.float32))

_tc3 = pl.pallas_call(
    _tc3_body, out_shape=jax.ShapeDtypeStruct((B, C), jnp.float32))


def kernel(x, edge_index, batch, W1, b1, W2, b2, Wfc1, bfc1, Wfc2, bfc2):
    src = edge_index[0]
    dst = edge_index[1]

    zeros16 = jnp.zeros((N_PAD, 16), jnp.float32)
    ones16 = jnp.ones((K_EDGE, 16), jnp.float32)
    degp = _deg_sc(dst, zeros16, ones16)

    h1s, dinv = _tc1(x, W1, degp)

    agg1 = _agg16(h1s, src, dst, zeros16)

    q = _tc2(agg1, h1s, dinv, b1.reshape(1, 16))

    agg2 = _agg16(q, src, dst, zeros16)

    return _tc3(agg2, q, dinv, b2.reshape(1, 32), W2, batch.reshape(1, N),
                Wfc1, bfc1.reshape(1, 64), Wfc2, bfc2.reshape(1, C))


# speedup vs baseline: 1.7512x; 1.0899x over previous
import functools

import jax
import jax.numpy as jnp
from jax import lax
from jax.experimental import pallas as pl
from jax.experimental.pallas import tpu as pltpu
from jax.experimental.pallas import tpu_sc as plsc

N = 10000
E = 320000
D = 128
B = 128
C = 10

NC = 2
NS = 16
NW = NC * NS
K_EDGE = 1000
AGG_NCH0, AGG_NCH1 = 11, 9
DEG_NCH0, DEG_NCH1 = 11, 9
E_CORE0 = NS * AGG_NCH0 * K_EDGE
E_DEG0 = NS * DEG_NCH0 * K_EDGE
N_PAD = 10240
RPT = N_PAD // NS


def _mesh():
    return plsc.VectorSubcoreMesh(core_axis_name="c", subcore_axis_name="s")


def _deg_body(dst_hbm, zeros_hbm, ones_hbm, out_hbm, *refs):
    nmax = max(DEG_NCH0, DEG_NCH1)
    idst = refs[:nmax]
    ones_v = refs[nmax]
    deg_sh = refs[nmax + 1]
    ssem = refs[nmax + 2]
    c = lax.axis_index("c")
    s = lax.axis_index("s")
    pltpu.sync_copy(zeros_hbm.at[pl.ds(s * RPT, RPT)],
                    deg_sh.at[pl.ds(s * RPT, RPT)])
    pltpu.sync_copy(ones_hbm, ones_v)

    def load(nch, base):
        for ch in range(nch):
            pltpu.sync_copy(
                dst_hbm.at[pl.ds(base + ch * K_EDGE, K_EDGE)], idst[ch])

    def scat(nch):
        for ch in range(nch):
            pltpu.async_copy(ones_v, deg_sh.at[idst[ch]], ssem, add=True)
        for ch in range(nch):
            pltpu.make_async_copy(ones_v, deg_sh.at[idst[ch]], ssem).wait()

    @pl.when(c == 0)
    def _():
        load(DEG_NCH0, pl.multiple_of(s * (DEG_NCH0 * K_EDGE), 8))

    @pl.when(c == 1)
    def _():
        load(DEG_NCH1, pl.multiple_of(E_DEG0 + s * (DEG_NCH1 * K_EDGE), 8))

    plsc.subcore_barrier()

    @pl.when(c == 0)
    def _():
        scat(DEG_NCH0)

    @pl.when(c == 1)
    def _():
        scat(DEG_NCH1)

    plsc.subcore_barrier()
    pltpu.sync_copy(deg_sh.at[pl.ds(s * RPT, RPT)],
                    out_hbm.at[c, pl.ds(s * RPT, RPT)])


_deg_sc = pl.kernel(
    _deg_body,
    out_type=jax.ShapeDtypeStruct((NC, N_PAD, 16), jnp.float32),
    mesh=_mesh(),
    scratch_types=(
        [pltpu.VMEM((K_EDGE,), jnp.int32)
         for _ in range(max(DEG_NCH0, DEG_NCH1))]
        + [pltpu.VMEM((K_EDGE, 16), jnp.float32),
           pltpu.VMEM_SHARED((N_PAD, 16), jnp.float32),
           pltpu.SemaphoreType.DMA]
    ),
    compiler_params=pltpu.CompilerParams(use_tc_tiling_on_sc=False),
)


def _agg_body(F, h_hbm, src_hbm, dst_hbm, zeros_hbm, out_hbm, *refs):
    nmax = max(AGG_NCH0, AGG_NCH1)
    isrc = refs[:nmax]
    idst = refs[nmax:2 * nmax]
    rows = refs[2 * nmax:2 * nmax + 2]
    agg_sh = refs[2 * nmax + 2]
    gsem = refs[2 * nmax + 3:2 * nmax + 5]
    ssem = refs[2 * nmax + 5:2 * nmax + 7]
    c = lax.axis_index("c")
    s = lax.axis_index("s")
    pltpu.sync_copy(zeros_hbm.at[pl.ds(s * RPT, RPT)],
                    agg_sh.at[pl.ds(s * RPT, RPT)])

    def load(nch, base):
        for ch in range(nch):
            pltpu.sync_copy(
                src_hbm.at[pl.ds(base + ch * K_EDGE, K_EDGE)], isrc[ch])
            pltpu.sync_copy(
                dst_hbm.at[pl.ds(base + ch * K_EDGE, K_EDGE)], idst[ch])

    def sg(ch, b):
        pltpu.async_copy(h_hbm.at[isrc[ch]], rows[b], gsem[b])

    def wg(ch, b):
        pltpu.make_async_copy(h_hbm.at[isrc[ch]], rows[b], gsem[b]).wait()

    def ss(ch, b):
        pltpu.async_copy(rows[b], agg_sh.at[idst[ch]], ssem[b], add=True)

    def ws(ch, b):
        pltpu.make_async_copy(rows[b], agg_sh.at[idst[ch]], ssem[b]).wait()

    def pipeline(nch):
        sg(0, 0)
        for ch in range(nch):
            b = ch % 2
            nb = (ch + 1) % 2
            wg(ch, b)
            ss(ch, b)
            if ch + 1 < nch:
                if ch >= 1:
                    ws(ch - 1, nb)
                sg(ch + 1, nb)
        if nch >= 2:
            ws(nch - 2, (nch - 2) % 2)
        ws(nch - 1, (nch - 1) % 2)

    @pl.when(c == 0)
    def _():
        load(AGG_NCH0, pl.multiple_of(s * (AGG_NCH0 * K_EDGE), 8))

    @pl.when(c == 1)
    def _():
        load(AGG_NCH1, pl.multiple_of(E_CORE0 + s * (AGG_NCH1 * K_EDGE), 8))

    plsc.subcore_barrier()

    @pl.when(c == 0)
    def _():
        pipeline(AGG_NCH0)

    @pl.when(c == 1)
    def _():
        pipeline(AGG_NCH1)

    plsc.subcore_barrier()
    pltpu.sync_copy(agg_sh.at[pl.ds(s * RPT, RPT)],
                    out_hbm.at[c, pl.ds(s * RPT, RPT)])


def _make_agg(F):
    nmax = max(AGG_NCH0, AGG_NCH1)
    return pl.kernel(
        functools.partial(_agg_body, F),
        out_type=jax.ShapeDtypeStruct((NC, N_PAD, F), jnp.float32),
        mesh=_mesh(),
        scratch_types=(
            [pltpu.VMEM((K_EDGE,), jnp.int32) for _ in range(2 * nmax)]
            + [pltpu.VMEM((K_EDGE, F), jnp.float32) for _ in range(2)]
            + [pltpu.VMEM_SHARED((N_PAD, F), jnp.float32)]
            + [pltpu.SemaphoreType.DMA for _ in range(4)]
        ),
        compiler_params=pltpu.CompilerParams(use_tc_tiling_on_sc=False),
    )


_agg16 = _make_agg(16)


def _tc1_body(x_ref, w1_ref, degp_ref, h1s_ref, dinv_ref):
    deg = degp_ref[0][:, :1] + degp_ref[1][:, :1] + 1.0
    dinv = lax.rsqrt(deg)
    dinv_ref[...] = dinv
    h = jnp.dot(x_ref[...], w1_ref[...], preferred_element_type=jnp.float32)
    h1s_ref[...] = h * dinv[:N]


def _tc2_body(agg_ref, h1s_ref, dinv_ref, b1_ref, q_ref):
    dinv = dinv_ref[...][:N]
    z = (agg_ref[0][:N] + agg_ref[1][:N] + h1s_ref[...]) * dinv + b1_ref[...]
    z = jnp.maximum(z, 0.0)
    q_ref[...] = z * dinv


def _tc3_body(agg_ref, q_ref, dinv_ref, b2_ref, w2_ref, batch_ref,
              wfc1_ref, bfc1_ref, wfc2_ref, bfc2_ref, out_ref):
    dinv = dinv_ref[...][:N]
    t = (agg_ref[0][:N] + agg_ref[1][:N] + q_ref[...]) * dinv
    z = jnp.dot(t, w2_ref[...],
                preferred_element_type=jnp.float32) + b2_ref[...]
    z = jnp.maximum(z, 0.0)
    oh = (batch_ref[...] == lax.broadcasted_iota(jnp.int32, (B, N), 0))
    oh = oh.astype(jnp.float32)
    sums = jnp.dot(oh, z, preferred_element_type=jnp.float32)
    counts = jnp.sum(oh, axis=1, keepdims=True)
    pooled = sums / jnp.maximum(counts, 1.0)
    hfc = jnp.maximum(
        jnp.dot(pooled, wfc1_ref[...], preferred_element_type=jnp.float32)
        + bfc1_ref[...], 0.0)
    out_ref[...] = jnp.dot(hfc, wfc2_ref[...],
                           preferred_element_type=jnp.float32) + bfc2_ref[...]


_tc1 = pl.pallas_call(
    _tc1_body,
    out_shape=(jax.ShapeDtypeStruct((N, 16), jnp.float32),
               jax.ShapeDtypeStruct((N_PAD, 1), jnp.float32)))

_tc2 = pl.pallas_call(
    _tc2_body, out_shape=jax.ShapeDtypeStruct((N, 16), jnp.float32))

_tc3 = pl.pallas_call(
    _tc3_body, out_shape=jax.ShapeDtypeStruct((B, C), jnp.float32))


def kernel(x, edge_index, batch, W1, b1, W2, b2, Wfc1, bfc1, Wfc2, bfc2):
    src = edge_index[0]
    dst = edge_index[1]

    zeros16 = jnp.zeros((N_PAD, 16), jnp.float32)
    ones16 = jnp.ones((K_EDGE, 16), jnp.float32)
    degp = _deg_sc(dst, zeros16, ones16)

    h1s, dinv = _tc1(x, W1, degp)

    agg1 = _agg16(h1s, src, dst, zeros16)

    q = _tc2(agg1, h1s, dinv, b1.reshape(1, 16))

    agg2 = _agg16(q, src, dst, zeros16)

    return _tc3(agg2, q, dinv, b2.reshape(1, 32), W2, batch.reshape(1, N),
                Wfc1, bfc1.reshape(1, 64), Wfc2, bfc2.reshape(1, C))
